# Initial kernel scaffold; baseline (speedup 1.0000x reference)
import jax
import jax.numpy as jnp
from jax.experimental import pallas as pl


def kernel(x):
    raise NotImplementedError("write your pallas kernel here")


# speedup vs baseline: 168.9238x; 168.9238x over previous
import functools

import jax
import jax.numpy as jnp
from jax import lax
from jax.experimental import pallas as pl
from jax.experimental.pallas import tpu as pltpu
from jax.experimental.pallas import tpu_sc as plsc

H = 512
W = 512
NSEG = 8
BAND = 5
NPOS = 2 * BAND + 1
LANES = 16
NPATH = NSEG - 1
NIMG = 8
NPROB = NIMG * 2 * NPATH
BIG = 3.0e8
GROW = H * LANES

_W0, _W1, _W2 = 0.2989, 0.587, 0.114


def _bands_body(x_ref, g_ref):
    xx = x_ref[0]
    gray = xx[0] * _W0 + xx[1] * _W1 + xx[2] * _W2
    zc = jnp.zeros((H, 1), dtype=jnp.float32)
    zr = jnp.zeros((1, W + 2), dtype=jnp.float32)
    g = jnp.concatenate([zr, jnp.concatenate([zc, gray, zc], axis=1), zr],
                        axis=0)
    gx = (g[:-2, 2:] + 2.0 * g[1:-1, 2:] + g[2:, 2:]) - (
        g[:-2, :-2] + 2.0 * g[1:-1, :-2] + g[2:, :-2])
    gy = (g[2:, :-2] + 2.0 * g[2:, 1:-1] + g[2:, 2:]) - (
        g[:-2, :-2] + 2.0 * g[:-2, 1:-1] + g[:-2, 2:])
    neg = -jnp.sqrt(gx * gx + gy * gy + 1e-08)
    negt = neg.T
    pad_c = jnp.full((H, LANES - NPOS), BIG, dtype=jnp.float32)
    gcs, grs = [], []
    for p in range(NPATH):
        b = (W // NSEG) * (p + 1) - BAND
        gcs.append(jnp.concatenate([neg[:, b:b + NPOS], pad_c], axis=1))
        grs.append(jnp.concatenate([negt[:, b:b + NPOS], pad_c], axis=1))
    g_ref[0] = jnp.stack([jnp.stack(gcs), jnp.stack(grs)])


_bands_call = pl.pallas_call(
    _bands_body,
    grid=(NIMG,),
    in_specs=[pl.BlockSpec((1, 3, H, W), lambda i: (i, 0, 0, 0))],
    out_specs=pl.BlockSpec((1, 2, NPATH, H, LANES),
                           lambda i: (i, 0, 0, 0, 0)),
    out_shape=jax.ShapeDtypeStruct((NIMG, 2, NPATH, H, LANES), jnp.float32),
)

_NC = 2
_NS = 16
_NW = _NC * _NS


def _lanegather(vec, idx):
    dnums = lax.GatherDimensionNumbers(
        offset_dims=(), collapsed_slice_dims=(0,), start_index_map=(0,))
    return lax.gather(vec, idx[:, None], dnums, slice_sizes=(1,),
                      mode=lax.GatherScatterMode.PROMISE_IN_BOUNDS)


def _sc_dp_body(g_hbm, pos_hbm, g_v, dec_v, pos_v):
    wid = lax.axis_index("s") * _NC + lax.axis_index("c")
    k_base = jnp.where(wid < 16, 4 * wid, 64 + 3 * (wid - 16))
    nmine = jnp.where(wid < 16, 4, 3)
    jj = lax.iota(jnp.int32, LANES)
    jL = jnp.maximum(jj - 1, 0)
    jR = jnp.minimum(jj + 1, NPOS - 1)

    for i in range(4):
        @pl.when(i < nmine)
        def _process():
            k = k_base + i
            rem = lax.rem(k, 2 * NPATH)
            p = lax.rem(rem, NPATH)
            pltpu.sync_copy(g_hbm.at[pl.ds(k * GROW, GROW)], g_v)
            cost0 = g_v[pl.ds(0, LANES)]

            def fstep(l, cost):
                cl = _lanegather(cost, jL)
                cr = _lanegather(cost, jR)
                m = jnp.minimum(jnp.minimum(cl, cost), cr)
                gv = g_v[pl.ds(l * LANES, LANES)]
                d = jnp.where((cl <= cost) & (cl <= cr), 0,
                              jnp.where(cost <= cr, 1, 2)).astype(jnp.int32)
                dec_v[pl.ds(l * LANES, LANES)] = d
                return m + gv

            costf = lax.fori_loop(1, H, fstep, cost0)
            mn = costf
            for s in (8, 4, 2, 1):
                mn = jnp.minimum(mn, _lanegather(mn, jj ^ s))
            cand = jnp.where(costf == mn, jj, LANES)
            for s in (8, 4, 2, 1):
                cand = jnp.minimum(cand, _lanegather(cand, jj ^ s))
            idx0 = cand
            base = (W // NSEG) * (p + 1) - BAND

            def bstep(t, carry):
                idx, acc = carry
                l = (H - 1) - t
                acc = jnp.where(jj == lax.rem(l, LANES), base + idx, acc)

                @pl.when(lax.rem(l, LANES) == 0)
                def _():
                    pos_v[pl.ds(i * H + l, LANES)] = acc

                d = _lanegather(dec_v[pl.ds(l * LANES, LANES)], idx)
                nidx = jnp.clip(idx + d - 1, 0, NPOS - 1)
                return nidx, acc

            idxe, acce = lax.fori_loop(
                0, H - 1, bstep, (idx0, jnp.zeros((LANES,), jnp.int32)))
            acce = jnp.where(jj == 0, base + idxe, acce)
            pos_v[pl.ds(i * H, LANES)] = acce
            pltpu.sync_copy(pos_v.at[pl.ds(i * H, H)],
                            pos_hbm.at[pl.ds(k * H, H)])


@functools.lru_cache(maxsize=None)
def _sc_dp_call():
    return functools.partial(
        pl.kernel,
        out_type=jax.ShapeDtypeStruct((NPROB * H,), jnp.int32),
        mesh=plsc.VectorSubcoreMesh(core_axis_name="c", subcore_axis_name="s",
                                    num_cores=_NC, num_subcores=_NS),
        scratch_types=[
            pltpu.VMEM((GROW,), jnp.float32),
            pltpu.VMEM((H * LANES,), jnp.int32),
            pltpu.VMEM((4 * H,), jnp.int32),
        ],
    )(_sc_dp_body)


def _labels_body(pos_ref, out_ref):
    vp = pos_ref[0, 0]
    hp = pos_ref[0, 1]
    vpt = vp.T
    r2 = lax.broadcasted_iota(jnp.int32, (H, W), 0)
    c2 = lax.broadcasted_iota(jnp.int32, (H, W), 1)
    acc = jnp.zeros((H, W), jnp.int32)
    for p in range(NPATH):
        acc = acc + (c2 >= vpt[:, p:p + 1]).astype(jnp.int32)
        acc = acc + 8 * (r2 >= hp[p:p + 1, :]).astype(jnp.int32)
    out_ref[0] = acc


_labels_call = pl.pallas_call(
    _labels_body,
    grid=(NIMG,),
    in_specs=[pl.BlockSpec((1, 2, NPATH, H), lambda i: (i, 0, 0, 0))],
    out_specs=pl.BlockSpec((1, H, W), lambda i: (i, 0, 0)),
    out_shape=jax.ShapeDtypeStruct((NIMG, H, W), jnp.int32),
)


def kernel(x):
    g = _bands_call(x)
    pos = _sc_dp_call()(g.reshape(-1))
    return _labels_call(pos.reshape(NIMG, 2, NPATH, H))


# speedup vs baseline: 192.9346x; 1.1421x over previous
import functools

import jax
import jax.numpy as jnp
from jax import lax
from jax.experimental import pallas as pl
from jax.experimental.pallas import tpu as pltpu
from jax.experimental.pallas import tpu_sc as plsc

H = 512
W = 512
NSEG = 8
BAND = 5
NPOS = 2 * BAND + 1
LANES = 16
NPATH = NSEG - 1
NIMG = 8
NPROB = NIMG * 2 * NPATH
BIG = 3.0e8
GROW = H * LANES
NCHAIN = 4

_W0, _W1, _W2 = 0.2989, 0.587, 0.114


def _bands_body(x_ref, g_ref):
    xx = x_ref[0]
    gray = xx[0] * _W0 + xx[1] * _W1 + xx[2] * _W2
    zc = jnp.zeros((H, 1), dtype=jnp.float32)
    zr = jnp.zeros((1, W + 2), dtype=jnp.float32)
    g = jnp.concatenate([zr, jnp.concatenate([zc, gray, zc], axis=1), zr],
                        axis=0)
    gx = (g[:-2, 2:] + 2.0 * g[1:-1, 2:] + g[2:, 2:]) - (
        g[:-2, :-2] + 2.0 * g[1:-1, :-2] + g[2:, :-2])
    gy = (g[2:, :-2] + 2.0 * g[2:, 1:-1] + g[2:, 2:]) - (
        g[:-2, :-2] + 2.0 * g[:-2, 1:-1] + g[:-2, 2:])
    neg = -jnp.sqrt(gx * gx + gy * gy + 1e-08)
    pad_c = jnp.full((H, LANES - NPOS), BIG, dtype=jnp.float32)
    lane2d = lax.broadcasted_iota(jnp.int32, (H, LANES), 1)
    gcs, grs = [], []
    for p in range(NPATH):
        b = (W // NSEG) * (p + 1) - BAND
        gcs.append(jnp.concatenate([neg[:, b:b + NPOS], pad_c], axis=1))
        strip = neg[b:b + LANES, :].T
        grs.append(jnp.where(lane2d < NPOS, strip, BIG))
    g_ref[0] = jnp.stack([jnp.stack(gcs), jnp.stack(grs)])


_bands_call = pl.pallas_call(
    _bands_body,
    grid=(NIMG,),
    in_specs=[pl.BlockSpec((1, 3, H, W), lambda i: (i, 0, 0, 0))],
    out_specs=pl.BlockSpec((1, 2, NPATH, H, LANES),
                           lambda i: (i, 0, 0, 0, 0)),
    out_shape=jax.ShapeDtypeStruct((NIMG, 2, NPATH, H, LANES), jnp.float32),
)

_NC = 2
_NS = 16
_NW = _NC * _NS


def _lanegather(vec, idx):
    dnums = lax.GatherDimensionNumbers(
        offset_dims=(), collapsed_slice_dims=(0,), start_index_map=(0,))
    return lax.gather(vec, idx[:, None], dnums, slice_sizes=(1,),
                      mode=lax.GatherScatterMode.PROMISE_IN_BOUNDS)


def _sc_dp_body(g_hbm, pos_hbm, g_v, dec_v, pos_v):
    wid = lax.axis_index("s") * _NC + lax.axis_index("c")
    jj = lax.iota(jnp.int32, LANES)
    jL = jnp.maximum(jj - 1, 0)
    jR = jnp.minimum(jj + 1, NPOS - 1)

    ks, valid = [], []
    for c in range(NCHAIN):
        kc = c * _NW + wid
        ks.append(jnp.minimum(kc, NPROB - 1))
        valid.append(kc < NPROB if c == NCHAIN - 1 else None)

    for c in range(NCHAIN):
        pltpu.sync_copy(g_hbm.at[pl.ds(ks[c] * GROW, GROW)],
                        g_v.at[pl.ds(c * GROW, GROW)])

    def fstep(l, costs):
        new = []
        for c in range(NCHAIN):
            cost = costs[c]
            cl = _lanegather(cost, jL)
            cr = _lanegather(cost, jR)
            m = jnp.minimum(jnp.minimum(cl, cost), cr)
            gv = g_v[pl.ds(c * GROW + l * LANES, LANES)]
            d = jnp.where((cl <= cost) & (cl <= cr), 0,
                          jnp.where(cost <= cr, 1, 2)).astype(jnp.int32)
            dec_v[pl.ds(c * GROW + l * LANES, LANES)] = d
            new.append(m + gv)
        return tuple(new)

    costs0 = tuple(g_v[pl.ds(c * GROW, LANES)] for c in range(NCHAIN))
    costsf = lax.fori_loop(1, H, fstep, costs0, unroll=2)

    idx0s, bases = [], []
    for c in range(NCHAIN):
        costf = costsf[c]
        mn = costf
        for s in (8, 4, 2, 1):
            mn = jnp.minimum(mn, _lanegather(mn, jj ^ s))
        cand = jnp.where(costf == mn, jj, LANES)
        for s in (8, 4, 2, 1):
            cand = jnp.minimum(cand, _lanegather(cand, jj ^ s))
        idx0s.append(cand)
        p = lax.rem(ks[c], NPATH)
        bases.append((W // NSEG) * (p + 1) - BAND)

    def bstep(t, carry):
        idxs, accs = carry
        l = (H - 1) - t
        lm = lax.rem(l, LANES)
        nidxs, naccs = [], []
        for c in range(NCHAIN):
            idx = idxs[c]
            naccs.append(jnp.where(jj == lm, bases[c] + idx, accs[c]))
            d = _lanegather(dec_v[pl.ds(c * GROW + l * LANES, LANES)], idx)
            nidxs.append(jnp.clip(idx + d - 1, 0, NPOS - 1))

        @pl.when(lm == 0)
        def _():
            for c in range(NCHAIN):
                pos_v[pl.ds(c * H + l, LANES)] = naccs[c]

        return tuple(nidxs), tuple(naccs)

    zacc = jnp.zeros((LANES,), jnp.int32)
    idxe, acce = lax.fori_loop(
        0, H - 1, bstep, (tuple(idx0s), (zacc,) * NCHAIN), unroll=2)
    for c in range(NCHAIN):
        first = jnp.where(jj == 0, bases[c] + idxe[c], acce[c])
        pos_v[pl.ds(c * H, LANES)] = first

    for c in range(NCHAIN):
        copy = lambda c=c: pltpu.sync_copy(
            pos_v.at[pl.ds(c * H, H)], pos_hbm.at[pl.ds(ks[c] * H, H)])
        if valid[c] is None:
            copy()
        else:
            pl.when(valid[c])(copy)


@functools.lru_cache(maxsize=None)
def _sc_dp_call():
    return functools.partial(
        pl.kernel,
        out_type=jax.ShapeDtypeStruct((NPROB * H,), jnp.int32),
        mesh=plsc.VectorSubcoreMesh(core_axis_name="c", subcore_axis_name="s",
                                    num_cores=_NC, num_subcores=_NS),
        scratch_types=[
            pltpu.VMEM((NCHAIN * GROW,), jnp.float32),
            pltpu.VMEM((NCHAIN * GROW,), jnp.int32),
            pltpu.VMEM((NCHAIN * H,), jnp.int32),
        ],
    )(_sc_dp_body)


def _labels_body(pos_ref, out_ref):
    vp = pos_ref[0, 0]
    hp = pos_ref[0, 1]
    vpt = vp.T
    r2 = lax.broadcasted_iota(jnp.int32, (H, W), 0)
    c2 = lax.broadcasted_iota(jnp.int32, (H, W), 1)
    acc = jnp.zeros((H, W), jnp.int32)
    for p in range(NPATH):
        acc = acc + (c2 >= vpt[:, p:p + 1]).astype(jnp.int32)
        acc = acc + 8 * (r2 >= hp[p:p + 1, :]).astype(jnp.int32)
    out_ref[0] = acc


_labels_call = pl.pallas_call(
    _labels_body,
    grid=(NIMG,),
    in_specs=[pl.BlockSpec((1, 2, NPATH, H), lambda i: (i, 0, 0, 0))],
    out_specs=pl.BlockSpec((1, H, W), lambda i: (i, 0, 0)),
    out_shape=jax.ShapeDtypeStruct((NIMG, H, W), jnp.int32),
)


def kernel(x):
    g = _bands_call(x)
    pos = _sc_dp_call()(g.reshape(-1))
    return _labels_call(pos.reshape(NIMG, 2, NPATH, H))


# speedup vs baseline: 204.2860x; 1.0588x over previous
import functools

import jax
import jax.numpy as jnp
from jax import lax
from jax.experimental import pallas as pl
from jax.experimental.pallas import tpu as pltpu
from jax.experimental.pallas import tpu_sc as plsc

H = 512
W = 512
NSEG = 8
BAND = 5
NPOS = 2 * BAND + 1
LANES = 16
NPATH = NSEG - 1
NIMG = 8
NPROB = NIMG * 2 * NPATH
BIG = 3.0e8
GROW = H * LANES
NCHAIN = 4

_W0, _W1, _W2 = 0.2989, 0.587, 0.114


def _bands_body(x_ref, g_ref):
    xx = x_ref[0]
    gray = xx[0] * _W0 + xx[1] * _W1 + xx[2] * _W2
    zc = jnp.zeros((H, 1), dtype=jnp.float32)
    zr = jnp.zeros((1, W + 2), dtype=jnp.float32)
    g = jnp.concatenate([zr, jnp.concatenate([zc, gray, zc], axis=1), zr],
                        axis=0)
    gx = (g[:-2, 2:] + 2.0 * g[1:-1, 2:] + g[2:, 2:]) - (
        g[:-2, :-2] + 2.0 * g[1:-1, :-2] + g[2:, :-2])
    gy = (g[2:, :-2] + 2.0 * g[2:, 1:-1] + g[2:, 2:]) - (
        g[:-2, :-2] + 2.0 * g[:-2, 1:-1] + g[:-2, 2:])
    neg = -jnp.sqrt(gx * gx + gy * gy + 1e-08)
    pad_c = jnp.full((H, LANES - NPOS), BIG, dtype=jnp.float32)
    lane2d = lax.broadcasted_iota(jnp.int32, (H, LANES), 1)
    gcs, grs = [], []
    for p in range(NPATH):
        b = (W // NSEG) * (p + 1) - BAND
        gcs.append(jnp.concatenate([neg[:, b:b + NPOS], pad_c], axis=1))
        strip = neg[b:b + LANES, :].T
        grs.append(jnp.where(lane2d < NPOS, strip, BIG))
    g_ref[0] = jnp.stack([jnp.stack(gcs), jnp.stack(grs)])


HIMG = NIMG // 2


def _make_bands_call(img_off):
    return pl.pallas_call(
        _bands_body,
        grid=(HIMG,),
        in_specs=[pl.BlockSpec((1, 3, H, W),
                               lambda i: (i + img_off, 0, 0, 0))],
        out_specs=pl.BlockSpec((1, 2, NPATH, H, LANES),
                               lambda i: (i, 0, 0, 0, 0)),
        out_shape=jax.ShapeDtypeStruct((HIMG, 2, NPATH, H, LANES),
                                       jnp.float32),
    )


_bands_calls = (_make_bands_call(0), _make_bands_call(HIMG))

_NC = 2
_NS = 16
_NW = _NC * _NS


def _lanegather(vec, idx):
    dnums = lax.GatherDimensionNumbers(
        offset_dims=(), collapsed_slice_dims=(0,), start_index_map=(0,))
    return lax.gather(vec, idx[:, None], dnums, slice_sizes=(1,),
                      mode=lax.GatherScatterMode.PROMISE_IN_BOUNDS)


def _sc_dp_body(g_hbm, pos_hbm, g_v, dec_v, pos_v, *, nprob, nchain):
    wid = lax.axis_index("s") * _NC + lax.axis_index("c")
    jj = lax.iota(jnp.int32, LANES)
    jL = jnp.maximum(jj - 1, 0)
    jR = jnp.minimum(jj + 1, NPOS - 1)

    ks, valid = [], []
    for c in range(nchain):
        kc = c * _NW + wid
        ks.append(jnp.minimum(kc, nprob - 1))
        valid.append(None if (c + 1) * _NW <= nprob else (kc < nprob))
    NCHAIN = nchain

    for c in range(NCHAIN):
        pltpu.sync_copy(g_hbm.at[pl.ds(ks[c] * GROW, GROW)],
                        g_v.at[pl.ds(c * GROW, GROW)])

    def fstep(l, costs):
        new = []
        for c in range(NCHAIN):
            cost = costs[c]
            cl = _lanegather(cost, jL)
            cr = _lanegather(cost, jR)
            m = jnp.minimum(jnp.minimum(cl, cost), cr)
            gv = g_v[pl.ds(c * GROW + l * LANES, LANES)]
            d = jnp.where((cl <= cost) & (cl <= cr), 0,
                          jnp.where(cost <= cr, 1, 2)).astype(jnp.int32)
            dec_v[pl.ds(c * GROW + l * LANES, LANES)] = d
            new.append(m + gv)
        return tuple(new)

    costs0 = tuple(g_v[pl.ds(c * GROW, LANES)] for c in range(NCHAIN))
    costsf = lax.fori_loop(1, H, fstep, costs0, unroll=2)

    idx0s, bases = [], []
    for c in range(NCHAIN):
        costf = costsf[c]
        mn = costf
        for s in (8, 4, 2, 1):
            mn = jnp.minimum(mn, _lanegather(mn, jj ^ s))
        cand = jnp.where(costf == mn, jj, LANES)
        for s in (8, 4, 2, 1):
            cand = jnp.minimum(cand, _lanegather(cand, jj ^ s))
        idx0s.append(cand)
        p = lax.rem(ks[c], NPATH)
        bases.append((W // NSEG) * (p + 1) - BAND)

    def bstep(t, carry):
        idxs, accs = carry
        l = (H - 1) - t
        lm = lax.rem(l, LANES)
        nidxs, naccs = [], []
        for c in range(NCHAIN):
            idx = idxs[c]
            naccs.append(jnp.where(jj == lm, bases[c] + idx, accs[c]))
            d = _lanegather(dec_v[pl.ds(c * GROW + l * LANES, LANES)], idx)
            nidxs.append(jnp.clip(idx + d - 1, 0, NPOS - 1))

        @pl.when(lm == 0)
        def _():
            for c in range(NCHAIN):
                pos_v[pl.ds(c * H + l, LANES)] = naccs[c]

        return tuple(nidxs), tuple(naccs)

    zacc = jnp.zeros((LANES,), jnp.int32)
    idxe, acce = lax.fori_loop(
        0, H - 1, bstep, (tuple(idx0s), (zacc,) * NCHAIN), unroll=2)
    for c in range(NCHAIN):
        first = jnp.where(jj == 0, bases[c] + idxe[c], acce[c])
        pos_v[pl.ds(c * H, LANES)] = first

    for c in range(NCHAIN):
        copy = lambda c=c: pltpu.sync_copy(
            pos_v.at[pl.ds(c * H, H)], pos_hbm.at[pl.ds(ks[c] * H, H)])
        if valid[c] is None:
            copy()
        else:
            pl.when(valid[c])(copy)


@functools.lru_cache(maxsize=None)
def _sc_dp_call(nprob):
    nchain = -(-nprob // _NW)
    body = functools.partial(_sc_dp_body, nprob=nprob, nchain=nchain)
    return functools.partial(
        pl.kernel,
        out_type=jax.ShapeDtypeStruct((nprob * H,), jnp.int32),
        mesh=plsc.VectorSubcoreMesh(core_axis_name="c", subcore_axis_name="s",
                                    num_cores=_NC, num_subcores=_NS),
        scratch_types=[
            pltpu.VMEM((nchain * GROW,), jnp.float32),
            pltpu.VMEM((nchain * GROW,), jnp.int32),
            pltpu.VMEM((nchain * H,), jnp.int32),
        ],
    )(body)


def _labels_body(pos_ref, out_ref):
    vp = pos_ref[0, 0]
    hp = pos_ref[0, 1]
    vpt = vp.T
    r2 = lax.broadcasted_iota(jnp.int32, (H, W), 0)
    c2 = lax.broadcasted_iota(jnp.int32, (H, W), 1)
    acc = jnp.zeros((H, W), jnp.int32)
    for p in range(NPATH):
        acc = acc + (c2 >= vpt[:, p:p + 1]).astype(jnp.int32)
        acc = acc + 8 * (r2 >= hp[p:p + 1, :]).astype(jnp.int32)
    out_ref[0] = acc


_labels_call = pl.pallas_call(
    _labels_body,
    grid=(NIMG,),
    in_specs=[pl.BlockSpec((1, 2, NPATH, H), lambda i: (i, 0, 0, 0))],
    out_specs=pl.BlockSpec((1, H, W), lambda i: (i, 0, 0)),
    out_shape=jax.ShapeDtypeStruct((NIMG, H, W), jnp.int32),
)


def kernel(x):
    hprob = HIMG * 2 * NPATH
    g1 = _bands_calls[0](x)
    pos1 = _sc_dp_call(hprob)(g1.reshape(-1))
    g2 = _bands_calls[1](x)
    pos2 = _sc_dp_call(hprob)(g2.reshape(-1))
    pos = jnp.concatenate([pos1.reshape(HIMG, 2, NPATH, H),
                           pos2.reshape(HIMG, 2, NPATH, H)])
    return _labels_call(pos)


# speedup vs baseline: 204.5251x; 1.0012x over previous
import functools

import jax
import jax.numpy as jnp
from jax import lax
from jax.experimental import pallas as pl
from jax.experimental.pallas import tpu as pltpu
from jax.experimental.pallas import tpu_sc as plsc

H = 512
W = 512
NSEG = 8
BAND = 5
NPOS = 2 * BAND + 1
LANES = 16
NPATH = NSEG - 1
NIMG = 8
NPROB = NIMG * 2 * NPATH
BIG = 3.0e8
GROW = H * LANES
NCHAIN = 4

_W0, _W1, _W2 = 0.2989, 0.587, 0.114


def _bands_body(x_ref, g_ref):
    xx = x_ref[0]
    gray = xx[0] * _W0 + xx[1] * _W1 + xx[2] * _W2
    zc = jnp.zeros((H, 1), dtype=jnp.float32)
    zr = jnp.zeros((1, W + 2), dtype=jnp.float32)
    g = jnp.concatenate([zr, jnp.concatenate([zc, gray, zc], axis=1), zr],
                        axis=0)
    gx = (g[:-2, 2:] + 2.0 * g[1:-1, 2:] + g[2:, 2:]) - (
        g[:-2, :-2] + 2.0 * g[1:-1, :-2] + g[2:, :-2])
    gy = (g[2:, :-2] + 2.0 * g[2:, 1:-1] + g[2:, 2:]) - (
        g[:-2, :-2] + 2.0 * g[:-2, 1:-1] + g[:-2, 2:])
    neg = -jnp.sqrt(gx * gx + gy * gy + 1e-08)
    pad_c = jnp.full((H, LANES - NPOS), BIG, dtype=jnp.float32)
    lane2d = lax.broadcasted_iota(jnp.int32, (H, LANES), 1)
    gcs, grs = [], []
    for p in range(NPATH):
        b = (W // NSEG) * (p + 1) - BAND
        gcs.append(jnp.concatenate([neg[:, b:b + NPOS], pad_c], axis=1))
        strip = neg[b:b + LANES, :].T
        grs.append(jnp.where(lane2d < NPOS, strip, BIG))
    g_ref[0] = jnp.stack([jnp.stack(gcs), jnp.stack(grs)])


HIMG = NIMG // 2


def _make_bands_call(img_off):
    return pl.pallas_call(
        _bands_body,
        grid=(HIMG,),
        in_specs=[pl.BlockSpec((1, 3, H, W),
                               lambda i: (i + img_off, 0, 0, 0))],
        out_specs=pl.BlockSpec((1, 2, NPATH, H, LANES),
                               lambda i: (i, 0, 0, 0, 0)),
        out_shape=jax.ShapeDtypeStruct((HIMG, 2, NPATH, H, LANES),
                                       jnp.float32),
    )


_bands_calls = (_make_bands_call(0), _make_bands_call(HIMG))

_NC = 2
_NS = 16
_NW = _NC * _NS


def _lanegather(vec, idx):
    dnums = lax.GatherDimensionNumbers(
        offset_dims=(), collapsed_slice_dims=(0,), start_index_map=(0,))
    return lax.gather(vec, idx[:, None], dnums, slice_sizes=(1,),
                      mode=lax.GatherScatterMode.PROMISE_IN_BOUNDS)


def _sc_dp_body(g_hbm, pos_hbm, g_v, dec_v, pos_v, *, nprob, nchain):
    wid = lax.axis_index("s") * _NC + lax.axis_index("c")
    jj = lax.iota(jnp.int32, LANES)
    jL = jnp.maximum(jj - 1, 0)
    jR = jnp.minimum(jj + 1, NPOS - 1)

    ks, valid = [], []
    for c in range(nchain):
        kc = c * _NW + wid
        ks.append(jnp.minimum(kc, nprob - 1))
        valid.append(None if (c + 1) * _NW <= nprob else (kc < nprob))
    NCHAIN = nchain

    for c in range(NCHAIN):
        pltpu.sync_copy(g_hbm.at[pl.ds(ks[c] * GROW, GROW)],
                        g_v.at[pl.ds(c * GROW, GROW)])

    def fstep(l, costs):
        new = []
        for c in range(NCHAIN):
            cost = costs[c]
            cl = _lanegather(cost, jL)
            cr = _lanegather(cost, jR)
            m = jnp.minimum(jnp.minimum(cl, cost), cr)
            gv = g_v[pl.ds(c * GROW + l * LANES, LANES)]
            d = jnp.where((cl <= cost) & (cl <= cr), 0,
                          jnp.where(cost <= cr, 1, 2)).astype(jnp.int32)
            dec_v[pl.ds(c * GROW + l * LANES, LANES)] = d
            new.append(m + gv)
        return tuple(new)

    costs0 = tuple(g_v[pl.ds(c * GROW, LANES)] for c in range(NCHAIN))
    costsf = lax.fori_loop(1, H, fstep, costs0, unroll=2)

    idx0s, bases = [], []
    for c in range(NCHAIN):
        costf = costsf[c]
        mn = costf
        for s in (8, 4, 2, 1):
            mn = jnp.minimum(mn, _lanegather(mn, jj ^ s))
        cand = jnp.where(costf == mn, jj, LANES)
        for s in (8, 4, 2, 1):
            cand = jnp.minimum(cand, _lanegather(cand, jj ^ s))
        idx0s.append(cand)
        p = lax.rem(ks[c], NPATH)
        bases.append((W // NSEG) * (p + 1) - BAND)

    def bstep(t, carry):
        idxs, accs = carry
        l = (H - 1) - t
        lm = lax.rem(l, LANES)
        nidxs, naccs = [], []
        for c in range(NCHAIN):
            idx = idxs[c]
            naccs.append(jnp.where(jj == lm, bases[c] + idx, accs[c]))
            d = _lanegather(dec_v[pl.ds(c * GROW + l * LANES, LANES)], idx)
            nidxs.append(jnp.clip(idx + d - 1, 0, NPOS - 1))

        @pl.when(lm == 0)
        def _():
            for c in range(NCHAIN):
                pos_v[pl.ds(c * H + l, LANES)] = naccs[c]

        return tuple(nidxs), tuple(naccs)

    zacc = jnp.zeros((LANES,), jnp.int32)
    idxe, acce = lax.fori_loop(
        0, H - 1, bstep, (tuple(idx0s), (zacc,) * NCHAIN), unroll=2)
    for c in range(NCHAIN):
        first = jnp.where(jj == 0, bases[c] + idxe[c], acce[c])
        pos_v[pl.ds(c * H, LANES)] = first

    for c in range(NCHAIN):
        copy = lambda c=c: pltpu.sync_copy(
            pos_v.at[pl.ds(c * H, H)], pos_hbm.at[pl.ds(ks[c] * H, H)])
        if valid[c] is None:
            copy()
        else:
            pl.when(valid[c])(copy)


@functools.lru_cache(maxsize=None)
def _sc_dp_call(nprob):
    nchain = -(-nprob // _NW)
    body = functools.partial(_sc_dp_body, nprob=nprob, nchain=nchain)
    return functools.partial(
        pl.kernel,
        out_type=jax.ShapeDtypeStruct((nprob * H,), jnp.int32),
        mesh=plsc.VectorSubcoreMesh(core_axis_name="c", subcore_axis_name="s",
                                    num_cores=_NC, num_subcores=_NS),
        scratch_types=[
            pltpu.VMEM((nchain * GROW,), jnp.float32),
            pltpu.VMEM((nchain * GROW,), jnp.int32),
            pltpu.VMEM((nchain * H,), jnp.int32),
        ],
    )(body)


def _labels_body(pos1_ref, pos2_ref, out_ref):
    half1 = pl.program_id(0) < HIMG
    sel = jnp.where(half1, pos1_ref[0], pos2_ref[0])
    vp = sel[0]
    hp = sel[1]
    vpt = vp.T
    r2 = lax.broadcasted_iota(jnp.int32, (H, W), 0)
    c2 = lax.broadcasted_iota(jnp.int32, (H, W), 1)
    acc = jnp.zeros((H, W), jnp.int32)
    for p in range(NPATH):
        acc = acc + (c2 >= vpt[:, p:p + 1]).astype(jnp.int32)
        acc = acc + 8 * (r2 >= hp[p:p + 1, :]).astype(jnp.int32)
    out_ref[0] = acc


_labels_call = pl.pallas_call(
    _labels_body,
    grid=(NIMG,),
    in_specs=[
        pl.BlockSpec((1, 2, NPATH, H),
                     lambda i: (jnp.minimum(i, HIMG - 1), 0, 0, 0)),
        pl.BlockSpec((1, 2, NPATH, H),
                     lambda i: (jnp.maximum(i - HIMG, 0), 0, 0, 0)),
    ],
    out_specs=pl.BlockSpec((1, H, W), lambda i: (i, 0, 0)),
    out_shape=jax.ShapeDtypeStruct((NIMG, H, W), jnp.int32),
)


def kernel(x):
    hprob = HIMG * 2 * NPATH
    g1 = _bands_calls[0](x)
    pos1 = _sc_dp_call(hprob)(g1.reshape(-1))
    g2 = _bands_calls[1](x)
    pos2 = _sc_dp_call(hprob)(g2.reshape(-1))
    return _labels_call(pos1.reshape(HIMG, 2, NPATH, H),
                        pos2.reshape(HIMG, 2, NPATH, H))


# speedup vs baseline: 205.4189x; 1.0044x over previous
import functools

import jax
import jax.numpy as jnp
from jax import lax
from jax.experimental import pallas as pl
from jax.experimental.pallas import tpu as pltpu
from jax.experimental.pallas import tpu_sc as plsc

H = 512
W = 512
NSEG = 8
BAND = 5
NPOS = 2 * BAND + 1
LANES = 16
NPATH = NSEG - 1
NIMG = 8
NPROB = NIMG * 2 * NPATH
BIG = 3.0e8
GROW = H * LANES
NCHAIN = 4

_W0, _W1, _W2 = 0.2989, 0.587, 0.114


def _bands_body(x_ref, g_ref):
    xx = x_ref[0]
    gray = xx[0] * _W0 + xx[1] * _W1 + xx[2] * _W2
    zc = jnp.zeros((H, 1), dtype=jnp.float32)
    zr = jnp.zeros((1, W + 2), dtype=jnp.float32)
    g = jnp.concatenate([zr, jnp.concatenate([zc, gray, zc], axis=1), zr],
                        axis=0)
    gx = (g[:-2, 2:] + 2.0 * g[1:-1, 2:] + g[2:, 2:]) - (
        g[:-2, :-2] + 2.0 * g[1:-1, :-2] + g[2:, :-2])
    gy = (g[2:, :-2] + 2.0 * g[2:, 1:-1] + g[2:, 2:]) - (
        g[:-2, :-2] + 2.0 * g[:-2, 1:-1] + g[:-2, 2:])
    neg = -jnp.sqrt(gx * gx + gy * gy + 1e-08)
    pad_c = jnp.full((H, LANES - NPOS), BIG, dtype=jnp.float32)
    lane2d = lax.broadcasted_iota(jnp.int32, (H, LANES), 1)
    gcs, grs = [], []
    for p in range(NPATH):
        b = (W // NSEG) * (p + 1) - BAND
        gcs.append(jnp.concatenate([neg[:, b:b + NPOS], pad_c], axis=1))
        strip = neg[b:b + LANES, :].T
        grs.append(jnp.where(lane2d < NPOS, strip, BIG))
    g_ref[0] = jnp.stack([jnp.stack(gcs), jnp.stack(grs)])


HIMG = NIMG // 2


def _make_bands_call(img_off):
    return pl.pallas_call(
        _bands_body,
        grid=(HIMG,),
        in_specs=[pl.BlockSpec((1, 3, H, W),
                               lambda i: (i + img_off, 0, 0, 0))],
        out_specs=pl.BlockSpec((1, 2, NPATH, H, LANES),
                               lambda i: (i, 0, 0, 0, 0)),
        out_shape=jax.ShapeDtypeStruct((HIMG, 2, NPATH, H, LANES),
                                       jnp.float32),
    )


_bands_calls = (_make_bands_call(0), _make_bands_call(HIMG))

_NC = 2
_NS = 16
_NW = _NC * _NS


def _lanegather(vec, idx):
    dnums = lax.GatherDimensionNumbers(
        offset_dims=(), collapsed_slice_dims=(0,), start_index_map=(0,))
    return lax.gather(vec, idx[:, None], dnums, slice_sizes=(1,),
                      mode=lax.GatherScatterMode.PROMISE_IN_BOUNDS)


def _sc_dp_body(g_hbm, pos_hbm, g_v, dec_v, pos_v, dma_sem, *, nprob, nchain):
    wid = lax.axis_index("s") * _NC + lax.axis_index("c")
    jj = lax.iota(jnp.int32, LANES)
    jL = jnp.maximum(jj - 1, 0)
    jR = jnp.minimum(jj + 1, NPOS - 1)

    ks, valid = [], []
    for c in range(nchain):
        kc = c * _NW + wid
        ks.append(jnp.minimum(kc, nprob - 1))
        valid.append(None if (c + 1) * _NW <= nprob else (kc < nprob))
    NCHAIN = nchain

    copies = [pltpu.async_copy(g_hbm.at[pl.ds(ks[c] * GROW, GROW)],
                               g_v.at[pl.ds(c * GROW, GROW)], dma_sem)
              for c in range(NCHAIN)]
    for cp in copies:
        cp.wait()

    def fstep(l, costs):
        new = []
        for c in range(NCHAIN):
            cost = costs[c]
            cl = _lanegather(cost, jL)
            cr = _lanegather(cost, jR)
            m = jnp.minimum(jnp.minimum(cl, cost), cr)
            gv = g_v[pl.ds(c * GROW + l * LANES, LANES)]
            d = jnp.where((cl <= cost) & (cl <= cr), 0,
                          jnp.where(cost <= cr, 1, 2)).astype(jnp.int32)
            dec_v[pl.ds(c * GROW + l * LANES, LANES)] = d
            new.append(m + gv)
        return tuple(new)

    costs0 = tuple(g_v[pl.ds(c * GROW, LANES)] for c in range(NCHAIN))
    costsf = lax.fori_loop(1, H, fstep, costs0, unroll=4)

    idx0s, bases = [], []
    for c in range(NCHAIN):
        costf = costsf[c]
        mn = costf
        for s in (8, 4, 2, 1):
            mn = jnp.minimum(mn, _lanegather(mn, jj ^ s))
        cand = jnp.where(costf == mn, jj, LANES)
        for s in (8, 4, 2, 1):
            cand = jnp.minimum(cand, _lanegather(cand, jj ^ s))
        idx0s.append(cand)
        p = lax.rem(ks[c], NPATH)
        bases.append((W // NSEG) * (p + 1) - BAND)

    def bstep(t, carry):
        idxs, accs = carry
        l = (H - 1) - t
        lm = lax.rem(l, LANES)
        nidxs, naccs = [], []
        for c in range(NCHAIN):
            idx = idxs[c]
            naccs.append(jnp.where(jj == lm, bases[c] + idx, accs[c]))
            d = _lanegather(dec_v[pl.ds(c * GROW + l * LANES, LANES)], idx)
            nidxs.append(jnp.clip(idx + d - 1, 0, NPOS - 1))

        @pl.when(lm == 0)
        def _():
            for c in range(NCHAIN):
                pos_v[pl.ds(c * H + l, LANES)] = naccs[c]

        return tuple(nidxs), tuple(naccs)

    zacc = jnp.zeros((LANES,), jnp.int32)
    idxe, acce = lax.fori_loop(
        0, H - 1, bstep, (tuple(idx0s), (zacc,) * NCHAIN), unroll=4)
    for c in range(NCHAIN):
        first = jnp.where(jj == 0, bases[c] + idxe[c], acce[c])
        pos_v[pl.ds(c * H, LANES)] = first

    for c in range(NCHAIN):
        copy = lambda c=c: pltpu.sync_copy(
            pos_v.at[pl.ds(c * H, H)], pos_hbm.at[pl.ds(ks[c] * H, H)])
        if valid[c] is None:
            copy()
        else:
            pl.when(valid[c])(copy)


@functools.lru_cache(maxsize=None)
def _sc_dp_call(nprob):
    nchain = -(-nprob // _NW)
    body = functools.partial(_sc_dp_body, nprob=nprob, nchain=nchain)
    return functools.partial(
        pl.kernel,
        out_type=jax.ShapeDtypeStruct((nprob * H,), jnp.int32),
        mesh=plsc.VectorSubcoreMesh(core_axis_name="c", subcore_axis_name="s",
                                    num_cores=_NC, num_subcores=_NS),
        scratch_types=[
            pltpu.VMEM((nchain * GROW,), jnp.float32),
            pltpu.VMEM((nchain * GROW,), jnp.int32),
            pltpu.VMEM((nchain * H,), jnp.int32),
            pltpu.SemaphoreType.DMA,
        ],
    )(body)


def _labels_body(pos1_ref, pos2_ref, out_ref):
    half1 = pl.program_id(0) < HIMG
    sel = jnp.where(half1, pos1_ref[0], pos2_ref[0])
    vp = sel[0]
    hp = sel[1]
    vpt = vp.T
    r2 = lax.broadcasted_iota(jnp.int32, (H, W), 0)
    c2 = lax.broadcasted_iota(jnp.int32, (H, W), 1)
    acc = jnp.zeros((H, W), jnp.int32)
    for p in range(NPATH):
        acc = acc + (c2 >= vpt[:, p:p + 1]).astype(jnp.int32)
        acc = acc + 8 * (r2 >= hp[p:p + 1, :]).astype(jnp.int32)
    out_ref[0] = acc


_labels_call = pl.pallas_call(
    _labels_body,
    grid=(NIMG,),
    in_specs=[
        pl.BlockSpec((1, 2, NPATH, H),
                     lambda i: (jnp.minimum(i, HIMG - 1), 0, 0, 0)),
        pl.BlockSpec((1, 2, NPATH, H),
                     lambda i: (jnp.maximum(i - HIMG, 0), 0, 0, 0)),
    ],
    out_specs=pl.BlockSpec((1, H, W), lambda i: (i, 0, 0)),
    out_shape=jax.ShapeDtypeStruct((NIMG, H, W), jnp.int32),
)


def kernel(x):
    hprob = HIMG * 2 * NPATH
    g1 = _bands_calls[0](x)
    pos1 = _sc_dp_call(hprob)(g1.reshape(-1))
    g2 = _bands_calls[1](x)
    pos2 = _sc_dp_call(hprob)(g2.reshape(-1))
    return _labels_call(pos1.reshape(HIMG, 2, NPATH, H),
                        pos2.reshape(HIMG, 2, NPATH, H))
